# baseline (speedup 1.0000x reference)
import jax
import jax.numpy as jnp
from jax.experimental import pallas as pl
from jax.experimental.pallas import tpu as pltpu

N = 2048
H = 128
E = 4
BJ = 256
BC = BJ // 128 * E
BI = 512
NT = BJ // 128
LS = BC * 128


def _dmpnn_body(adj_ref, e_ref, h_ref, WT_ref, UT_ref, B_ref,
                Wb_ref, Ub_ref, out_ref, acc_h, acc_s, acc_d):
    j = pl.program_id(0)
    i = pl.program_id(1)
    ni = pl.num_programs(1)

    @pl.when(i == 0)
    def _init():
        acc_h[:] = jnp.zeros_like(acc_h)
        acc_s[:] = jnp.zeros_like(acc_s)
        acc_d[:] = jnp.zeros_like(acc_d)

    m = adj_ref[:].astype(jnp.float32)
    h_i = h_ref[pl.ds(i * BI, BI), :]

    acc_h[:] += jax.lax.dot_general(
        m, h_i, (((0,), (0,)), ((), ())),
        preferred_element_type=jnp.float32)
    acc_d[:] += jnp.sum(m, axis=0, keepdims=True)

    m3 = jnp.broadcast_to(
        m.reshape(BI, NT, 1, 128), (BI, NT, E, 128)).reshape(BI, BC, 128)
    acc_s[:] += jnp.sum(e_ref[:] * m3, axis=0)

    @pl.when(i == ni - 1)
    def _finish():
        G = jnp.concatenate([acc_s[:].reshape(1, LS), acc_d[:]], axis=1)
        lanes = jax.lax.broadcasted_iota(jnp.int32, (BJ, LS + BJ), 1)
        rows = jax.lax.broadcasted_iota(jnp.int32, (BJ, LS + BJ), 0)
        jsel = jnp.where(lanes < LS,
                         (lanes // (E * 128)) * 128 + lanes % 128,
                         lanes - LS)
        DG = jnp.where(jsel == rows, G, 0.0)

        WhT = WT_ref[:H, :]
        msg = (acc_h[:] @ WhT
               + jax.lax.dot(DG, B_ref[:],
                             preferred_element_type=jnp.float32))
        h_j = h_ref[pl.ds(j * BJ, BJ), :]
        out_ref[:] = (h_j + msg) @ UT_ref[:] + Ub_ref[:]


def kernel(h, edge_attr, adj, W_w, W_b, U_w, U_b):
    e3 = edge_attr.reshape(N, N // 128, 128, E)
    e3 = e3.transpose(0, 1, 3, 2).reshape(N, (N // 128) * E, 128)
    WT = W_w.T
    UT = U_w.T
    WeT = WT[H:H + E, :]
    B = jnp.concatenate(
        [jnp.tile(jnp.repeat(WeT, 128, axis=0), (NT, 1)),
         jnp.broadcast_to(W_b[None, :], (BJ, H))], axis=0)
    Wb = W_b.reshape(1, H)
    Ub = U_b.reshape(1, H)

    grid = (N // BJ, N // BI)
    out = pl.pallas_call(
        _dmpnn_body,
        grid=grid,
        in_specs=[
            pl.BlockSpec((BI, BJ), lambda j, i: (i, j)),
            pl.BlockSpec((BI, BC, 128), lambda j, i: (i, j, 0)),
            pl.BlockSpec((N, H), lambda j, i: (0, 0)),
            pl.BlockSpec((H + E, H), lambda j, i: (0, 0)),
            pl.BlockSpec((H, H), lambda j, i: (0, 0)),
            pl.BlockSpec((LS + BJ, H), lambda j, i: (0, 0)),
            pl.BlockSpec((1, H), lambda j, i: (0, 0)),
            pl.BlockSpec((1, H), lambda j, i: (0, 0)),
        ],
        out_specs=pl.BlockSpec((BJ, H), lambda j, i: (j, 0)),
        out_shape=jax.ShapeDtypeStruct((N, H), jnp.float32),
        scratch_shapes=[
            pltpu.VMEM((BJ, H), jnp.float32),
            pltpu.VMEM((BC, 128), jnp.float32),
            pltpu.VMEM((1, BJ), jnp.float32),
        ],
        compiler_params=pltpu.CompilerParams(
            dimension_semantics=("parallel", "arbitrary")),
    )(adj, e3, h, WT, UT, B, Wb, Ub)
    return out


# speedup vs baseline: 1.2225x; 1.2225x over previous
import jax
import jax.numpy as jnp
from jax.experimental import pallas as pl
from jax.experimental.pallas import tpu as pltpu

N = 2048
H = 128
E = 4
BJ = 512
BC = BJ // 128 * E
BI = 512
NT = BJ // 128
LS = BC * 128


def _dmpnn_body(adj_ref, e_ref, h_ref, WT_ref, UT_ref, B_ref,
                Wb_ref, Ub_ref, out_ref, acc_h, acc_s, acc_d):
    j = pl.program_id(0)
    i = pl.program_id(1)
    ni = pl.num_programs(1)

    @pl.when(i == 0)
    def _init():
        acc_h[:] = jnp.zeros_like(acc_h)
        acc_s[:] = jnp.zeros_like(acc_s)
        acc_d[:] = jnp.zeros_like(acc_d)

    m = adj_ref[:].astype(jnp.float32)
    h_i = h_ref[pl.ds(i * BI, BI), :]

    acc_h[:] += jax.lax.dot_general(
        m, h_i, (((0,), (0,)), ((), ())),
        preferred_element_type=jnp.float32)
    acc_d[:] += jnp.sum(m, axis=0, keepdims=True)

    m3 = jnp.broadcast_to(
        m.reshape(BI, NT, 1, 128), (BI, NT, E, 128)).reshape(BI, BC, 128)
    acc_s[:] += jnp.sum(e_ref[:] * m3, axis=0)

    @pl.when(i == ni - 1)
    def _finish():
        G = jnp.concatenate([acc_s[:].reshape(1, LS), acc_d[:]], axis=1)
        lanes = jax.lax.broadcasted_iota(jnp.int32, (BJ, LS + BJ), 1)
        rows = jax.lax.broadcasted_iota(jnp.int32, (BJ, LS + BJ), 0)
        jsel = jnp.where(lanes < LS,
                         (lanes // (E * 128)) * 128 + lanes % 128,
                         lanes - LS)
        DG = jnp.where(jsel == rows, G, 0.0)

        WhT = WT_ref[:H, :]
        msg = (acc_h[:] @ WhT
               + jax.lax.dot(DG, B_ref[:],
                             preferred_element_type=jnp.float32))
        h_j = h_ref[pl.ds(j * BJ, BJ), :]
        out_ref[:] = (h_j + msg) @ UT_ref[:] + Ub_ref[:]


def kernel(h, edge_attr, adj, W_w, W_b, U_w, U_b):
    e3 = edge_attr.reshape(N, N // 128, 128, E)
    e3 = e3.transpose(0, 1, 3, 2).reshape(N, (N // 128) * E, 128)
    WT = W_w.T
    UT = U_w.T
    WeT = WT[H:H + E, :]
    B = jnp.concatenate(
        [jnp.tile(jnp.repeat(WeT, 128, axis=0), (NT, 1)),
         jnp.broadcast_to(W_b[None, :], (BJ, H))], axis=0)
    Wb = W_b.reshape(1, H)
    Ub = U_b.reshape(1, H)

    grid = (N // BJ, N // BI)
    out = pl.pallas_call(
        _dmpnn_body,
        grid=grid,
        in_specs=[
            pl.BlockSpec((BI, BJ), lambda j, i: (i, j)),
            pl.BlockSpec((BI, BC, 128), lambda j, i: (i, j, 0)),
            pl.BlockSpec((N, H), lambda j, i: (0, 0)),
            pl.BlockSpec((H + E, H), lambda j, i: (0, 0)),
            pl.BlockSpec((H, H), lambda j, i: (0, 0)),
            pl.BlockSpec((LS + BJ, H), lambda j, i: (0, 0)),
            pl.BlockSpec((1, H), lambda j, i: (0, 0)),
            pl.BlockSpec((1, H), lambda j, i: (0, 0)),
        ],
        out_specs=pl.BlockSpec((BJ, H), lambda j, i: (j, 0)),
        out_shape=jax.ShapeDtypeStruct((N, H), jnp.float32),
        scratch_shapes=[
            pltpu.VMEM((BJ, H), jnp.float32),
            pltpu.VMEM((BC, 128), jnp.float32),
            pltpu.VMEM((1, BJ), jnp.float32),
        ],
        compiler_params=pltpu.CompilerParams(
            dimension_semantics=("parallel", "arbitrary")),
    )(adj, e3, h, WT, UT, B, Wb, Ub)
    return out
